# baseline (speedup 1.0000x reference)
import jax
import jax.numpy as jnp
from jax import lax
from jax.experimental import pallas as pl
from jax.experimental.pallas import tpu as pltpu
from jax.experimental.pallas import tpu_sc as plsc

_IN_CH = 16
_OUT_CH = 32
_ROW = 32
_COL = 32
_ROW_P = 34
_COL_P = 34
_NPIX = _ROW * _COL
_IMG_WORDS = _IN_CH * _ROW * _COL
_PAD_WORDS = _IN_CH * _ROW_P * _COL_P
_OUT_WORDS = _OUT_CH * _NPIX
_NC = 2
_NS = 16
_REGS = 16
_PASSES = _NPIX // (_REGS * 16)


def _sc_body(img_hbm, table_hbm, starts_hbm, bias_hbm, out_hbm,
             stage, img, acc, table_v, starts_v, bias_v,
             starts_s, bias_s):
    ntab = table_hbm.shape[0]
    cid = lax.axis_index("c")
    sid = lax.axis_index("s")
    b = sid * _NC + cid

    pltpu.sync_copy(img_hbm.at[b], stage)
    pltpu.sync_copy(table_hbm, table_v.at[pl.ds(0, ntab)])
    pltpu.sync_copy(starts_hbm, starts_v.at[pl.ds(0, _OUT_CH + 1)])
    pltpu.sync_copy(bias_hbm, bias_v)

    sv = [starts_v[pl.ds(0, 16)], starts_v[pl.ds(16, 16)],
          starts_v[pl.ds(32, 16)]]
    for i in range(_OUT_CH + 1):
        starts_s[i] = sv[i // 16][i % 16]
    bv = [bias_v[pl.ds(0, 16)], bias_v[pl.ds(16, 16)]]
    for i in range(_OUT_CH):
        bias_s[i] = bv[i // 16][i % 16]

    zero = jnp.zeros((16,), jnp.float32)

    @plsc.parallel_loop(0, _PAD_WORDS // 16, unroll=8)
    def _zero_pad(i):
        img[pl.ds(i * 16, 16)] = zero

    @plsc.parallel_loop(0, _IN_CH * _ROW, unroll=4)
    def _relayout(i):
        ch = i >> 5
        r = i & 31
        dst = ch * (_ROW_P * _COL_P) + (r + 1) * _COL_P + 1
        img[pl.ds(dst, 16)] = stage[ch, r, pl.ds(0, 16)]
        img[pl.ds(dst + 16, 16)] = stage[ch, r, pl.ds(16, 16)]

    iota = lax.iota(jnp.int32, 16)

    @pl.loop(0, _OUT_CH)
    def _chan(oc):
        lo = starts_s[oc]
        hi = starts_s[oc + 1]
        bvec = jnp.full((16,), bias_s[oc], jnp.float32)
        for ps in range(_PASSES):
            piota = iota + ps * (_NPIX // _PASSES // _COL) * _COL_P

            def _wbody(l, accs):
                t16 = table_v[pl.ds(l * 2, 16)]
                tf = plsc.bitcast(t16, jnp.float32)
                cur = piota + t16[0]
                w = tf[1]
                out = []
                for k in range(_REGS):
                    g = plsc.load_gather(img, [cur])
                    out.append(accs[k] + g * w)
                    if k + 1 < _REGS:
                        cur = cur + (16 if (k & 1) == 0 else _COL_P - 16)
                return tuple(out)

            accs = pl.loop(lo, hi, init_carry=(bvec,) * _REGS)(_wbody)
            for k in range(_REGS):
                acc[pl.ds(oc * _NPIX + ps * _REGS * 16 + k * 16, 16)] = accs[k]

    pltpu.sync_copy(acc, out_hbm.at[b])


def kernel(images, weight_value, bias_value, image_weight_index,
           image_range_flat, weight_oc_ids, bias_index):
    del image_range_flat
    bsz = images.shape[0]
    nw = weight_value.shape[0]
    assert bsz == _NC * _NS

    widx = image_weight_index.astype(jnp.int32)
    table = jnp.stack(
        [widx, jax.lax.bitcast_convert_type(weight_value, jnp.int32)],
        axis=1).reshape(-1)
    starts = jnp.sum(weight_oc_ids[None, :] < jnp.arange(_OUT_CH + 1)[:, None],
                     axis=1).astype(jnp.int32)
    onehot = (bias_index[:, None] == jnp.arange(_OUT_CH)[None, :])
    bias_full = jnp.sum(jnp.where(onehot, bias_value[:, None], 0.0), axis=0)

    mesh = plsc.VectorSubcoreMesh(core_axis_name="c", subcore_axis_name="s",
                                  num_cores=_NC, num_subcores=_NS)
    conv = pl.kernel(
        _sc_body,
        out_type=jax.ShapeDtypeStruct((bsz, _OUT_WORDS), jnp.float32),
        mesh=mesh,
        compiler_params=pltpu.CompilerParams(needs_layout_passes=False),
        scratch_types=[
            pltpu.VMEM((_IN_CH, _ROW, _COL), jnp.float32),
            pltpu.VMEM((_PAD_WORDS,), jnp.float32),
            pltpu.VMEM((_OUT_WORDS,), jnp.float32),
            pltpu.VMEM((2 * nw + 16,), jnp.int32),
            pltpu.VMEM((48,), jnp.int32),
            pltpu.VMEM((_OUT_CH,), jnp.float32),
            pltpu.SMEM((_OUT_CH + 8,), jnp.int32),
            pltpu.SMEM((_OUT_CH,), jnp.float32),
        ],
    )
    out = conv(images, table, starts, bias_full)
    return out.reshape(bsz, _OUT_CH, _ROW, _COL)


# speedup vs baseline: 1.0304x; 1.0304x over previous
import jax
import jax.numpy as jnp
from jax import lax
from jax.experimental import pallas as pl
from jax.experimental.pallas import tpu as pltpu
from jax.experimental.pallas import tpu_sc as plsc

_IN_CH = 16
_OUT_CH = 32
_ROW = 32
_COL = 32
_ROW_P = 34
_COL_P = 34
_NPIX = _ROW * _COL
_IMG_WORDS = _IN_CH * _ROW * _COL
_PAD_WORDS = _IN_CH * _ROW_P * _COL_P
_OUT_WORDS = _OUT_CH * _NPIX
_NC = 2
_NS = 16
_REGS = 16
_PASSES = _NPIX // (_REGS * 16)


def _sc_body(img_hbm, widx_hbm, wval_hbm, starts_hbm, bias_hbm, out_hbm,
             stage, img, acc, widx_v, wval_v, starts_v, bias_v,
             starts_s, bias_s):
    nw = widx_hbm.shape[0]
    cid = lax.axis_index("c")
    sid = lax.axis_index("s")
    b = sid * _NC + cid

    pltpu.sync_copy(img_hbm.at[b], stage)
    pltpu.sync_copy(widx_hbm, widx_v.at[pl.ds(0, nw)])
    pltpu.sync_copy(wval_hbm, wval_v.at[pl.ds(0, nw)])
    pltpu.sync_copy(starts_hbm, starts_v.at[pl.ds(0, _OUT_CH + 1)])
    pltpu.sync_copy(bias_hbm, bias_v)

    sv = [starts_v[pl.ds(0, 16)], starts_v[pl.ds(16, 16)],
          starts_v[pl.ds(32, 16)]]
    for i in range(_OUT_CH + 1):
        starts_s[i] = sv[i // 16][i % 16]
    bv = [bias_v[pl.ds(0, 16)], bias_v[pl.ds(16, 16)]]
    for i in range(_OUT_CH):
        bias_s[i] = bv[i // 16][i % 16]

    zero = jnp.zeros((16,), jnp.float32)

    @plsc.parallel_loop(0, _PAD_WORDS // 16, unroll=8)
    def _zero_pad(i):
        img[pl.ds(i * 16, 16)] = zero

    @plsc.parallel_loop(0, _IN_CH * _ROW, unroll=4)
    def _relayout(i):
        ch = i >> 5
        r = i & 31
        dst = ch * (_ROW_P * _COL_P) + (r + 1) * _COL_P + 1
        img[pl.ds(dst, 16)] = stage[ch, r, pl.ds(0, 16)]
        img[pl.ds(dst + 16, 16)] = stage[ch, r, pl.ds(16, 16)]

    iota = lax.iota(jnp.int32, 16)

    @pl.loop(0, _OUT_CH)
    def _chan(oc):
        lo = starts_s[oc]
        hi = starts_s[oc + 1]
        bvec = jnp.full((16,), bias_s[oc], jnp.float32)
        for ps in range(_PASSES):
            piota = iota + ps * (_NPIX // _PASSES // _COL) * _COL_P

            def _wbody(l, accs):
                iv16 = widx_v[pl.ds(l, 16)]
                wv16 = wval_v[pl.ds(l, 16)]
                cur = piota + iv16[0]
                w = wv16[0]
                out = []
                for k in range(_REGS):
                    g = plsc.load_gather(img, [cur])
                    out.append(accs[k] + g * w)
                    if k + 1 < _REGS:
                        cur = cur + (16 if (k & 1) == 0 else _COL_P - 16)
                return tuple(out)

            accs = pl.loop(lo, hi, init_carry=(bvec,) * _REGS)(_wbody)
            for k in range(_REGS):
                acc[pl.ds(oc * _NPIX + ps * _REGS * 16 + k * 16, 16)] = accs[k]

    pltpu.sync_copy(acc, out_hbm.at[b])


def kernel(images, weight_value, bias_value, image_weight_index,
           image_range_flat, weight_oc_ids, bias_index):
    del image_range_flat
    bsz = images.shape[0]
    nw = weight_value.shape[0]
    assert bsz == _NC * _NS

    widx = image_weight_index.astype(jnp.int32)
    starts = jnp.sum(weight_oc_ids[None, :] < jnp.arange(_OUT_CH + 1)[:, None],
                     axis=1).astype(jnp.int32)
    onehot = (bias_index[:, None] == jnp.arange(_OUT_CH)[None, :])
    bias_full = jnp.sum(jnp.where(onehot, bias_value[:, None], 0.0), axis=0)

    mesh = plsc.VectorSubcoreMesh(core_axis_name="c", subcore_axis_name="s",
                                  num_cores=_NC, num_subcores=_NS)
    conv = pl.kernel(
        _sc_body,
        out_type=jax.ShapeDtypeStruct((bsz, _OUT_WORDS), jnp.float32),
        mesh=mesh,
        compiler_params=pltpu.CompilerParams(needs_layout_passes=False),
        scratch_types=[
            pltpu.VMEM((_IN_CH, _ROW, _COL), jnp.float32),
            pltpu.VMEM((_PAD_WORDS,), jnp.float32),
            pltpu.VMEM((_OUT_WORDS,), jnp.float32),
            pltpu.VMEM((nw + 16,), jnp.int32),
            pltpu.VMEM((nw + 16,), jnp.float32),
            pltpu.VMEM((48,), jnp.int32),
            pltpu.VMEM((_OUT_CH,), jnp.float32),
            pltpu.SMEM((_OUT_CH + 8,), jnp.int32),
            pltpu.SMEM((_OUT_CH,), jnp.float32),
        ],
    )
    out = conv(images, widx, weight_value, starts, bias_full)
    return out.reshape(bsz, _OUT_CH, _ROW, _COL)


# speedup vs baseline: 1.0338x; 1.0033x over previous
import jax
import jax.numpy as jnp
from jax import lax
from jax.experimental import pallas as pl
from jax.experimental.pallas import tpu as pltpu
from jax.experimental.pallas import tpu_sc as plsc

_IN_CH = 16
_OUT_CH = 32
_ROW = 32
_COL = 32
_ROW_P = 34
_COL_P = 34
_NPIX = _ROW * _COL
_IMG_WORDS = _IN_CH * _ROW * _COL
_PAD_WORDS = _IN_CH * _ROW_P * _COL_P
_OUT_WORDS = _OUT_CH * _NPIX
_NC = 2
_NS = 16
_REGS = 16
_PASSES = _NPIX // (_REGS * 16)


def _sc_body(img_hbm, widx_hbm, wval_hbm, starts_hbm, bias_hbm, out_hbm,
             stage, img, acc, widx_v, wval_v, starts_v, bias_v,
             starts_s, bias_s):
    nw = widx_hbm.shape[0]
    cid = lax.axis_index("c")
    sid = lax.axis_index("s")
    b = sid * _NC + cid

    pltpu.sync_copy(img_hbm.at[b], stage)
    pltpu.sync_copy(widx_hbm, widx_v.at[pl.ds(0, nw)])
    pltpu.sync_copy(wval_hbm, wval_v.at[pl.ds(0, nw)])
    pltpu.sync_copy(starts_hbm, starts_v.at[pl.ds(0, _OUT_CH + 1)])
    pltpu.sync_copy(bias_hbm, bias_v)

    sv = [starts_v[pl.ds(0, 16)], starts_v[pl.ds(16, 16)],
          starts_v[pl.ds(32, 16)]]
    for i in range(_OUT_CH + 1):
        starts_s[i] = sv[i // 16][i % 16]
    bv = [bias_v[pl.ds(0, 16)], bias_v[pl.ds(16, 16)]]
    for i in range(_OUT_CH):
        bias_s[i] = bv[i // 16][i % 16]

    zero = jnp.zeros((16,), jnp.float32)

    @plsc.parallel_loop(0, _PAD_WORDS // 16, unroll=8)
    def _zero_pad(i):
        img[pl.ds(i * 16, 16)] = zero

    @plsc.parallel_loop(0, _IN_CH * _ROW, unroll=4)
    def _relayout(i):
        ch = i >> 5
        r = i & 31
        dst = ch * (_ROW_P * _COL_P) + (r + 1) * _COL_P + 1
        img[pl.ds(dst, 16)] = stage[ch, r, pl.ds(0, 16)]
        img[pl.ds(dst + 16, 16)] = stage[ch, r, pl.ds(16, 16)]

    iota = lax.iota(jnp.int32, 16)

    @pl.loop(0, _OUT_CH)
    def _chan(oc):
        lo = starts_s[oc]
        hi = starts_s[oc + 1]
        bvec = jnp.full((16,), bias_s[oc], jnp.float32)
        for ps in range(_PASSES):
            piota = iota + ps * (_NPIX // _PASSES // _COL) * _COL_P

            def _wbody(l, accs):
                iv16 = widx_v[pl.ds(l, 16)]
                wv16 = wval_v[pl.ds(l, 16)]
                cur = piota + iv16[0]
                w = wv16[0]
                out = []
                for k in range(_REGS):
                    g = plsc.load_gather(img, [cur])
                    out.append(accs[k] + g * w)
                    if k + 1 < _REGS:
                        cur = cur + (16 if (k & 1) == 0 else _COL_P - 16)
                return tuple(out)

            accs = pl.loop(lo, hi, init_carry=(bvec,) * _REGS)(_wbody)
            for k in range(_REGS):
                acc[pl.ds(oc * _NPIX + ps * _REGS * 16 + k * 16, 16)] = accs[k]

    pltpu.sync_copy(acc, out_hbm.at[b])


def kernel(images, weight_value, bias_value, image_weight_index,
           image_range_flat, weight_oc_ids, bias_index):
    del image_range_flat
    bsz = images.shape[0]
    nw = weight_value.shape[0]
    assert bsz == _NC * _NS

    widx = image_weight_index.astype(jnp.int32)
    starts = jnp.sum(weight_oc_ids[None, :] < jnp.arange(_OUT_CH + 1)[:, None],
                     axis=1).astype(jnp.int32)
    onehot = (bias_index[:, None] == jnp.arange(_OUT_CH)[None, :])
    bias_full = jnp.sum(jnp.where(onehot, bias_value[:, None], 0.0), axis=0)

    mesh = plsc.VectorSubcoreMesh(core_axis_name="c", subcore_axis_name="s",
                                  num_cores=_NC, num_subcores=_NS)
    conv = pl.kernel(
        _sc_body,
        out_type=jax.ShapeDtypeStruct((bsz, _OUT_WORDS), jnp.float32),
        mesh=mesh,
        compiler_params=pltpu.CompilerParams(needs_layout_passes=False,
                                             skip_device_barrier=True),
        scratch_types=[
            pltpu.VMEM((_IN_CH, _ROW, _COL), jnp.float32),
            pltpu.VMEM((_PAD_WORDS,), jnp.float32),
            pltpu.VMEM((_OUT_WORDS,), jnp.float32),
            pltpu.VMEM((nw + 16,), jnp.int32),
            pltpu.VMEM((nw + 16,), jnp.float32),
            pltpu.VMEM((48,), jnp.int32),
            pltpu.VMEM((_OUT_CH,), jnp.float32),
            pltpu.SMEM((_OUT_CH + 8,), jnp.int32),
            pltpu.SMEM((_OUT_CH,), jnp.float32),
        ],
    )
    out = conv(images, widx, weight_value, starts, bias_full)
    return out.reshape(bsz, _OUT_CH, _ROW, _COL)
